# baseline (speedup 1.0000x reference)
import jax
import jax.numpy as jnp
from jax.experimental import pallas as pl
from jax.experimental.pallas import tpu as pltpu


def _affine_dense_kernel(w_ref, b_ref, x_ref, o_ref):
    x0 = x_ref[0, 0]
    x1 = x_ref[1, 0]
    o_ref[0] = x0 * w_ref[0, 0] + x1 * w_ref[0, 1] + b_ref[0]


def _affine_narrow_kernel(w_ref, b_ref, x_ref, o_ref):
    x0 = x_ref[:, 0:1]
    x1 = x_ref[:, 1:2]
    o_ref[...] = x0 * w_ref[0, 0] + x1 * w_ref[0, 1] + b_ref[0]


def _narrow_path(xf, weight, bias):
    B = xf.shape[0]
    tile = 16384
    while tile > 8 and B % tile != 0:
        tile //= 2
    if B % tile != 0:
        tile = B
    return pl.pallas_call(
        _affine_narrow_kernel,
        out_shape=jax.ShapeDtypeStruct((B, 1), jnp.float32),
        grid=(B // tile,),
        in_specs=[
            pl.BlockSpec(memory_space=pltpu.MemorySpace.SMEM),
            pl.BlockSpec(memory_space=pltpu.MemorySpace.SMEM),
            pl.BlockSpec((tile, 2), lambda i: (i, 0)),
        ],
        out_specs=pl.BlockSpec((tile, 1), lambda i: (i, 0)),
        compiler_params=pltpu.CompilerParams(
            dimension_semantics=("parallel",),
        ),
    )(weight, bias, xf)


def kernel(x, weight, bias):
    B = x.shape[0]
    xf = x.astype(jnp.float32)
    wf = weight.astype(jnp.float32)
    bf = bias.astype(jnp.float32)

    lo = 131072
    while lo > 128 and B % (8 * lo) != 0:
        lo //= 2
    if B % (8 * lo) != 0:
        return _narrow_path(xf, wf, bf)
    nb = B // (8 * lo)

    xv = xf.T.reshape(2, nb, 8, lo)
    out = pl.pallas_call(
        _affine_dense_kernel,
        out_shape=jax.ShapeDtypeStruct((nb, 8, lo), jnp.float32),
        grid=(nb,),
        in_specs=[
            pl.BlockSpec(memory_space=pltpu.MemorySpace.SMEM),
            pl.BlockSpec(memory_space=pltpu.MemorySpace.SMEM),
            pl.BlockSpec((2, 1, 8, lo), lambda i: (0, i, 0, 0)),
        ],
        out_specs=pl.BlockSpec((1, 8, lo), lambda i: (i, 0, 0)),
        compiler_params=pltpu.CompilerParams(
            dimension_semantics=("parallel",),
        ),
    )(wf, bf, xv)
    return out.reshape(B, 1)


# speedup vs baseline: 3.3732x; 3.3732x over previous
import jax
import jax.numpy as jnp
from jax.experimental import pallas as pl
from jax.experimental.pallas import tpu as pltpu


def _affine_lane_kernel(w_ref, b_ref, x_ref, o_ref):
    o_ref[...] = (x_ref[0:1, :] * w_ref[0, 0]
                  + x_ref[1:2, :] * w_ref[0, 1] + b_ref[0])


def _affine_narrow_kernel(w_ref, b_ref, x_ref, o_ref):
    x0 = x_ref[:, 0:1]
    x1 = x_ref[:, 1:2]
    o_ref[...] = x0 * w_ref[0, 0] + x1 * w_ref[0, 1] + b_ref[0]


def _narrow_path(xf, weight, bias):
    B = xf.shape[0]
    tile = 16384
    while tile > 8 and B % tile != 0:
        tile //= 2
    if B % tile != 0:
        tile = B
    return pl.pallas_call(
        _affine_narrow_kernel,
        out_shape=jax.ShapeDtypeStruct((B, 1), jnp.float32),
        grid=(B // tile,),
        in_specs=[
            pl.BlockSpec(memory_space=pltpu.MemorySpace.SMEM),
            pl.BlockSpec(memory_space=pltpu.MemorySpace.SMEM),
            pl.BlockSpec((tile, 2), lambda i: (i, 0)),
        ],
        out_specs=pl.BlockSpec((tile, 1), lambda i: (i, 0)),
        compiler_params=pltpu.CompilerParams(
            dimension_semantics=("parallel",),
        ),
    )(weight, bias, xf)


def kernel(x, weight, bias):
    B = x.shape[0]
    xf = x.astype(jnp.float32)
    wf = weight.astype(jnp.float32)
    bf = bias.astype(jnp.float32)

    lanes = 131072
    while lanes > 128 and B % lanes != 0:
        lanes //= 2
    if B % lanes != 0:
        return _narrow_path(xf, wf, bf)
    n = B // lanes

    xt = xf.T
    yt = pl.pallas_call(
        _affine_lane_kernel,
        out_shape=jax.ShapeDtypeStruct((1, B), jnp.float32),
        grid=(n,),
        in_specs=[
            pl.BlockSpec(memory_space=pltpu.MemorySpace.SMEM),
            pl.BlockSpec(memory_space=pltpu.MemorySpace.SMEM),
            pl.BlockSpec((2, lanes), lambda i: (0, i)),
        ],
        out_specs=pl.BlockSpec((1, lanes), lambda i: (0, i)),
        compiler_params=pltpu.CompilerParams(
            dimension_semantics=("parallel",),
        ),
    )(wf, bf, xt)
    return yt.reshape(B, 1)


# speedup vs baseline: 4.6922x; 1.3910x over previous
import jax
import jax.numpy as jnp
from jax.experimental import pallas as pl
from jax.experimental.pallas import tpu as pltpu


def _affine_lane_kernel(w_ref, b_ref, x_ref, o_ref):
    o_ref[...] = (x_ref[0:1, :] * w_ref[0, 0]
                  + x_ref[1:2, :] * w_ref[0, 1] + b_ref[0])


def _affine_narrow_kernel(w_ref, b_ref, x_ref, o_ref):
    x0 = x_ref[:, 0:1]
    x1 = x_ref[:, 1:2]
    o_ref[...] = x0 * w_ref[0, 0] + x1 * w_ref[0, 1] + b_ref[0]


def _narrow_path(xf, weight, bias):
    B = xf.shape[0]
    tile = 16384
    while tile > 8 and B % tile != 0:
        tile //= 2
    if B % tile != 0:
        tile = B
    return pl.pallas_call(
        _affine_narrow_kernel,
        out_shape=jax.ShapeDtypeStruct((B, 1), jnp.float32),
        grid=(B // tile,),
        in_specs=[
            pl.BlockSpec(memory_space=pltpu.MemorySpace.SMEM),
            pl.BlockSpec(memory_space=pltpu.MemorySpace.SMEM),
            pl.BlockSpec((tile, 2), lambda i: (i, 0)),
        ],
        out_specs=pl.BlockSpec((tile, 1), lambda i: (i, 0)),
        compiler_params=pltpu.CompilerParams(
            dimension_semantics=("parallel",),
        ),
    )(weight, bias, xf)


def kernel(x, weight, bias):
    B = x.shape[0]
    xf = x.astype(jnp.float32)
    wf = weight.astype(jnp.float32)
    bf = bias.astype(jnp.float32)

    lanes = 262144
    while lanes > 128 and B % lanes != 0:
        lanes //= 2
    if B % lanes != 0:
        return _narrow_path(xf, wf, bf)
    n = B // lanes

    xt = xf.T
    yt = pl.pallas_call(
        _affine_lane_kernel,
        out_shape=jax.ShapeDtypeStruct((1, B), jnp.float32),
        grid=(n,),
        in_specs=[
            pl.BlockSpec(memory_space=pltpu.MemorySpace.SMEM),
            pl.BlockSpec(memory_space=pltpu.MemorySpace.SMEM),
            pl.BlockSpec((2, lanes), lambda i: (0, i)),
        ],
        out_specs=pl.BlockSpec((1, lanes), lambda i: (0, i)),
        compiler_params=pltpu.CompilerParams(
            dimension_semantics=("parallel",),
        ),
    )(wf, bf, xt)
    return yt.reshape(B, 1)


# speedup vs baseline: 5.4441x; 1.1602x over previous
import jax
import jax.numpy as jnp
from jax.experimental import pallas as pl
from jax.experimental.pallas import tpu as pltpu


def _affine_lane_kernel(w_ref, b_ref, x_ref, o_ref):
    o_ref[...] = (x_ref[0:1, :] * w_ref[0, 0]
                  + x_ref[1:2, :] * w_ref[0, 1] + b_ref[0])


def _affine_narrow_kernel(w_ref, b_ref, x_ref, o_ref):
    x0 = x_ref[:, 0:1]
    x1 = x_ref[:, 1:2]
    o_ref[...] = x0 * w_ref[0, 0] + x1 * w_ref[0, 1] + b_ref[0]


def _narrow_path(xf, weight, bias):
    B = xf.shape[0]
    tile = 16384
    while tile > 8 and B % tile != 0:
        tile //= 2
    if B % tile != 0:
        tile = B
    return pl.pallas_call(
        _affine_narrow_kernel,
        out_shape=jax.ShapeDtypeStruct((B, 1), jnp.float32),
        grid=(B // tile,),
        in_specs=[
            pl.BlockSpec(memory_space=pltpu.MemorySpace.SMEM),
            pl.BlockSpec(memory_space=pltpu.MemorySpace.SMEM),
            pl.BlockSpec((tile, 2), lambda i: (i, 0)),
        ],
        out_specs=pl.BlockSpec((tile, 1), lambda i: (i, 0)),
        compiler_params=pltpu.CompilerParams(
            dimension_semantics=("parallel",),
        ),
    )(weight, bias, xf)


def kernel(x, weight, bias):
    B = x.shape[0]
    xf = x.astype(jnp.float32)
    wf = weight.astype(jnp.float32)
    bf = bias.astype(jnp.float32)

    lanes = 524288
    while lanes > 128 and B % lanes != 0:
        lanes //= 2
    if B % lanes != 0:
        return _narrow_path(xf, wf, bf)
    n = B // lanes

    xt = xf.T
    yt = pl.pallas_call(
        _affine_lane_kernel,
        out_shape=jax.ShapeDtypeStruct((1, B), jnp.float32),
        grid=(n,),
        in_specs=[
            pl.BlockSpec(memory_space=pltpu.MemorySpace.SMEM),
            pl.BlockSpec(memory_space=pltpu.MemorySpace.SMEM),
            pl.BlockSpec((2, lanes), lambda i: (0, i)),
        ],
        out_specs=pl.BlockSpec((1, lanes), lambda i: (0, i)),
        compiler_params=pltpu.CompilerParams(
            dimension_semantics=("parallel",),
        ),
    )(wf, bf, xt)
    return yt.reshape(B, 1)


# speedup vs baseline: 5.6304x; 1.0342x over previous
import jax
import jax.numpy as jnp
from jax.experimental import pallas as pl
from jax.experimental.pallas import tpu as pltpu


def _affine_lane_kernel(w_ref, b_ref, x_ref, o_ref):
    o_ref[...] = (x_ref[0:1, :] * w_ref[0, 0]
                  + x_ref[1:2, :] * w_ref[0, 1] + b_ref[0])


def _affine_narrow_kernel(w_ref, b_ref, x_ref, o_ref):
    x0 = x_ref[:, 0:1]
    x1 = x_ref[:, 1:2]
    o_ref[...] = x0 * w_ref[0, 0] + x1 * w_ref[0, 1] + b_ref[0]


def _narrow_path(xf, weight, bias):
    B = xf.shape[0]
    tile = 16384
    while tile > 8 and B % tile != 0:
        tile //= 2
    if B % tile != 0:
        tile = B
    return pl.pallas_call(
        _affine_narrow_kernel,
        out_shape=jax.ShapeDtypeStruct((B, 1), jnp.float32),
        grid=(B // tile,),
        in_specs=[
            pl.BlockSpec(memory_space=pltpu.MemorySpace.SMEM),
            pl.BlockSpec(memory_space=pltpu.MemorySpace.SMEM),
            pl.BlockSpec((tile, 2), lambda i: (i, 0)),
        ],
        out_specs=pl.BlockSpec((tile, 1), lambda i: (i, 0)),
        compiler_params=pltpu.CompilerParams(
            dimension_semantics=("parallel",),
        ),
    )(weight, bias, xf)


def kernel(x, weight, bias):
    B = x.shape[0]
    xf = x.astype(jnp.float32)
    wf = weight.astype(jnp.float32)
    bf = bias.astype(jnp.float32)

    lanes = 1048576
    while lanes > 128 and B % lanes != 0:
        lanes //= 2
    if B % lanes != 0:
        return _narrow_path(xf, wf, bf)
    n = B // lanes

    xt = xf.T
    yt = pl.pallas_call(
        _affine_lane_kernel,
        out_shape=jax.ShapeDtypeStruct((1, B), jnp.float32),
        grid=(n,),
        in_specs=[
            pl.BlockSpec(memory_space=pltpu.MemorySpace.SMEM),
            pl.BlockSpec(memory_space=pltpu.MemorySpace.SMEM),
            pl.BlockSpec((2, lanes), lambda i: (0, i)),
        ],
        out_specs=pl.BlockSpec((1, lanes), lambda i: (0, i)),
        compiler_params=pltpu.CompilerParams(
            dimension_semantics=("parallel",),
        ),
    )(wf, bf, xt)
    return yt.reshape(B, 1)
